# baseline (device time: 24623 ns/iter reference)
import jax
import jax.numpy as jnp
from jax import lax
from jax.experimental import pallas as pl
from jax.experimental.pallas import tpu as pltpu

N_DEV = 32
N_STEPS = 3
N_SLOTS = 7
SLOTS = {"x": [0], "y": [1, 2, 3], "z": [4, 5, 6]}
A_ORDER = ("x", "y", "z")
B_ORDER = ("y", "z", "x")


def kernel(x, router_W, route_idx, expert_W, shared_W):
    n, d = x.shape
    h = expert_W.shape[-1]
    half = n // 2

    def body(x_ref, rw_ref, idx_ref, ew_ref, sw_ref, out_ref,
             acc_ref, recv_a, recv_b,
             send_sems_a, recv_sems_a, send_sems_b, recv_sems_b):
        my_i = lax.axis_index("i")

        def y_partner(p, dy):
            z = p >> 3
            s = p & 7
            y = s >> 1
            px = (s & 1) ^ (y & 1)
            ny = y ^ dy
            ns = (ny << 1) | (px ^ (ny & 1))
            return (z << 3) | ns

        def step_partners(kind, p):
            if kind == "x":
                return [p ^ 1]
            if kind == "y":
                return [y_partner(p, dy) for dy in (1, 2, 3)]
            return [p ^ 8, p ^ 16, p ^ 24]

        barrier_sem = pltpu.get_barrier_semaphore()
        n_partners = 0
        for kind in A_ORDER:
            for ptn in step_partners(kind, my_i):
                pl.semaphore_signal(
                    barrier_sem, inc=1,
                    device_id=(ptn,), device_id_type=pl.DeviceIdType.MESH,
                )
                n_partners += 1

        xv = x_ref[:, :]

        scores = jnp.dot(xv, rw_ref[:, :], preferred_element_type=jnp.float32)
        s_max = jnp.max(scores, axis=-1, keepdims=True)
        e = jnp.exp(scores - s_max)
        probs = e / jnp.sum(e, axis=-1, keepdims=True)
        idx = idx_ref[:, :]
        eids = lax.broadcasted_iota(jnp.int32, scores.shape, 1)
        p_sel = jnp.sum(jnp.where(eids == idx, probs, 0.0),
                        axis=-1, keepdims=True)
        w0 = jnp.where(idx == 2 * my_i, p_sel, 0.0)
        w1 = jnp.where(idx == 2 * my_i + 1, p_sel, 0.0)

        def partial_half(lo):
            xh = xv[lo:lo + half, :]
            y0 = jnp.dot(xh, ew_ref[0], preferred_element_type=jnp.float32)
            y1 = jnp.dot(xh, ew_ref[1], preferred_element_type=jnp.float32)
            return (w0[lo:lo + half] * y0
                    + w1[lo:lo + half] * y1).astype(jnp.bfloat16)

        def make_rdmas(kind, lo, recv, ssems, rsems):
            return [
                pltpu.make_async_remote_copy(
                    src_ref=acc_ref.at[pl.ds(lo, half)],
                    dst_ref=recv.at[slot],
                    send_sem=ssems.at[slot],
                    recv_sem=rsems.at[slot],
                    device_id=(ptn,),
                    device_id_type=pl.DeviceIdType.MESH,
                )
                for slot, ptn in zip(
                    SLOTS[kind], step_partners(kind, my_i))
            ]

        def recv_sum(kind, recv):
            acc = recv[SLOTS[kind][0]]
            for slot in SLOTS[kind][1:]:
                acc = acc + recv[slot]
            return acc

        for t in range(N_STEPS):
            ras = make_rdmas(A_ORDER[t], 0, recv_a, send_sems_a, recv_sems_a)
            rbs = make_rdmas(B_ORDER[t], half, recv_b,
                             send_sems_b, recv_sems_b)
            if t == 0:
                acc_ref[pl.ds(0, half), :] = partial_half(0)
                pl.semaphore_wait(barrier_sem, n_partners)
                for r in ras:
                    r.start()
                acc_ref[pl.ds(half, half), :] = partial_half(half)
                for r in rbs:
                    r.start()
                out_ref[:, :] = jnp.dot(
                    xv, sw_ref[:, :], preferred_element_type=jnp.float32)
            else:
                for r in ras:
                    r.start()
                for r in rbs:
                    r.start()
            for r in ras:
                r.wait()
            if t < N_STEPS - 1:
                acc_ref[pl.ds(0, half), :] = (
                    acc_ref[pl.ds(0, half), :] + recv_sum(A_ORDER[t], recv_a))
            else:
                out_ref[pl.ds(0, half), :] = (
                    out_ref[pl.ds(0, half), :]
                    + (acc_ref[pl.ds(0, half), :]
                       + recv_sum(A_ORDER[t], recv_a)).astype(jnp.float32))
            for r in rbs:
                r.wait()
            if t < N_STEPS - 1:
                acc_ref[pl.ds(half, half), :] = (
                    acc_ref[pl.ds(half, half), :]
                    + recv_sum(B_ORDER[t], recv_b))
            else:
                out_ref[pl.ds(half, half), :] = (
                    out_ref[pl.ds(half, half), :]
                    + (acc_ref[pl.ds(half, half), :]
                       + recv_sum(B_ORDER[t], recv_b)).astype(jnp.float32))

    return pl.pallas_call(
        body,
        out_shape=jax.ShapeDtypeStruct((n, h), jnp.float32),
        in_specs=[pl.BlockSpec(memory_space=pltpu.VMEM)] * 5,
        out_specs=pl.BlockSpec(memory_space=pltpu.VMEM),
        scratch_shapes=[
            pltpu.VMEM((n, h), jnp.bfloat16),
            pltpu.VMEM((N_SLOTS, n // 2, h), jnp.bfloat16),
            pltpu.VMEM((N_SLOTS, n // 2, h), jnp.bfloat16),
            pltpu.SemaphoreType.DMA((N_SLOTS,)),
            pltpu.SemaphoreType.DMA((N_SLOTS,)),
            pltpu.SemaphoreType.DMA((N_SLOTS,)),
            pltpu.SemaphoreType.DMA((N_SLOTS,)),
        ],
        compiler_params=pltpu.CompilerParams(collective_id=0),
    )(x, router_W, route_idx, expert_W, shared_W)
